# baseline (speedup 1.0000x reference)
import jax
import jax.numpy as jnp
from jax.experimental import pallas as pl
from jax.experimental.pallas import tpu as pltpu

T = 2048
D = 768
E = 16
FM = 256
G = 8
KG = 4
SCALE = 2.5
TT = 512

_DOT_PREC = jax.lax.Precision.DEFAULT


def _dot(a, b):
    return jax.lax.dot_general(a, b, (((1,), (1,)), ((), ())),
                               precision=_DOT_PREC,
                               preferred_element_type=jnp.float32)


def _moe_kernel(x_ref, gate_w_ref, gate_b_ref, w1_hbm, w2_hbm, w3_hbm,
                sw1_ref, sw2_ref, sw3_ref, out_ref, w1s, w2s, w3s, sems):
    t = pl.program_id(0)

    def _copy(which, src, dst, e):
        return pltpu.make_async_copy(src.at[e], dst.at[e], sems.at[which, e])

    @pl.when(t == 0)
    def _start_dmas():
        for e in range(E):
            _copy(0, w1_hbm, w1s, e).start()
            _copy(2, w3_hbm, w3s, e).start()
            _copy(1, w2_hbm, w2s, e).start()

    x = x_ref[...]

    scores_t = jax.nn.sigmoid(_dot(gate_w_ref[...], x))
    sb_t = scores_t + gate_b_ref[...]
    gs_t = sb_t.reshape(G, 2, TT).sum(axis=1)
    ga = gs_t[:, None, :]
    gb = gs_t[None, :, :]
    gidx = jax.lax.broadcasted_iota(jnp.int32, (G, G, TT), 0)
    oidx = jax.lax.broadcasted_iota(jnp.int32, (G, G, TT), 1)
    beats = jnp.logical_or(gb > ga,
                           jnp.logical_and(gb == ga, oidx < gidx))
    rank = jnp.where(beats, 1.0, 0.0).sum(axis=1)
    sel_g = jnp.where(rank < KG, 1.0, 0.0)
    sel_e = jnp.broadcast_to(sel_g[:, None, :], (G, 2, TT)).reshape(E, TT)
    w = sel_e * scores_t
    denom = w.sum(axis=0, keepdims=True)
    cw = (w * (SCALE / denom)).T

    hs = jax.nn.silu(_dot(x, sw1_ref[...])) * _dot(x, sw3_ref[...])
    acc = _dot(hs, sw2_ref[...])

    for e in range(E):
        @pl.when(t == 0)
        def _wait13():
            _copy(0, w1_hbm, w1s, e).wait()
            _copy(2, w3_hbm, w3s, e).wait()

        h1 = _dot(x, w1s[e])
        h3 = _dot(x, w3s[e])
        h = jax.nn.silu(h1) * h3 * cw[:, e:e + 1]

        @pl.when(t == 0)
        def _wait2():
            _copy(1, w2_hbm, w2s, e).wait()

        acc += _dot(h, w2s[e])

    out_ref[...] = acc


@jax.jit
def kernel(x, gate_w, gate_b, W1, W2, W3, sw1, sw2, sw3):
    return pl.pallas_call(
        _moe_kernel,
        grid=(T // TT,),
        in_specs=[
            pl.BlockSpec((TT, D), lambda t: (t, 0)),
            pl.BlockSpec((E, D), lambda t: (0, 0)),
            pl.BlockSpec((E, 1), lambda t: (0, 0)),
            pl.BlockSpec(memory_space=pltpu.MemorySpace.HBM),
            pl.BlockSpec(memory_space=pltpu.MemorySpace.HBM),
            pl.BlockSpec(memory_space=pltpu.MemorySpace.HBM),
            pl.BlockSpec((FM, D), lambda t: (0, 0)),
            pl.BlockSpec((D, FM), lambda t: (0, 0)),
            pl.BlockSpec((FM, D), lambda t: (0, 0)),
        ],
        out_specs=pl.BlockSpec((TT, D), lambda t: (t, 0)),
        out_shape=jax.ShapeDtypeStruct((T, D), x.dtype),
        scratch_shapes=[
            pltpu.VMEM((E, FM, D), jnp.float32),
            pltpu.VMEM((E, D, FM), jnp.float32),
            pltpu.VMEM((E, FM, D), jnp.float32),
            pltpu.SemaphoreType.DMA((3, E)),
        ],
    )(x, gate_w, gate_b.reshape(E, 1), W1, W2, W3, sw1, sw2, sw3)


# speedup vs baseline: 1.2831x; 1.2831x over previous
import jax
import jax.numpy as jnp
from jax.experimental import pallas as pl
from jax.experimental.pallas import tpu as pltpu

T = 2048
D = 768
E = 16
FM = 256
G = 8
KG = 4
SCALE = 2.5
TT = 512

_DOT_PREC = jax.lax.Precision.DEFAULT


def _dot(a, b):
    return jax.lax.dot_general(a, b, (((1,), (1,)), ((), ())),
                               precision=_DOT_PREC,
                               preferred_element_type=jnp.float32)


def _moe_kernel(x_ref, gate_w_ref, gate_b_ref, w1_ref, w2_ref, w3_ref,
                sw1_ref, sw2_ref, sw3_ref, out_ref):
    x = x_ref[...]

    scores_t = jax.nn.sigmoid(_dot(gate_w_ref[...], x))
    sb_t = scores_t + gate_b_ref[...]
    gs_t = sb_t.reshape(G, 2, TT).sum(axis=1)
    ga = gs_t[:, None, :]
    gb = gs_t[None, :, :]
    gidx = jax.lax.broadcasted_iota(jnp.int32, (G, G, TT), 0)
    oidx = jax.lax.broadcasted_iota(jnp.int32, (G, G, TT), 1)
    beats = jnp.logical_or(gb > ga,
                           jnp.logical_and(gb == ga, oidx < gidx))
    rank = jnp.where(beats, 1.0, 0.0).sum(axis=1)
    sel_g = jnp.where(rank < KG, 1.0, 0.0)
    sel_e = jnp.broadcast_to(sel_g[:, None, :], (G, 2, TT)).reshape(E, TT)
    w = sel_e * scores_t
    denom = w.sum(axis=0, keepdims=True)
    cw = (w * (SCALE / denom)).T

    hs = jax.nn.silu(_dot(x, sw1_ref[...])) * _dot(x, sw3_ref[...])
    acc = _dot(hs, sw2_ref[...])

    for e in range(E):
        h1 = _dot(x, w1_ref[e])
        h3 = _dot(x, w3_ref[e])
        h = jax.nn.silu(h1) * h3 * cw[:, e:e + 1]
        acc += _dot(h, w2_ref[e])

    out_ref[...] = acc


@jax.jit
def kernel(x, gate_w, gate_b, W1, W2, W3, sw1, sw2, sw3):
    return pl.pallas_call(
        _moe_kernel,
        grid=(T // TT,),
        in_specs=[
            pl.BlockSpec((TT, D), lambda t: (t, 0)),
            pl.BlockSpec((E, D), lambda t: (0, 0)),
            pl.BlockSpec((E, 1), lambda t: (0, 0)),
            pl.BlockSpec((E, FM, D), lambda t: (0, 0, 0)),
            pl.BlockSpec((E, D, FM), lambda t: (0, 0, 0)),
            pl.BlockSpec((E, FM, D), lambda t: (0, 0, 0)),
            pl.BlockSpec((FM, D), lambda t: (0, 0)),
            pl.BlockSpec((D, FM), lambda t: (0, 0)),
            pl.BlockSpec((FM, D), lambda t: (0, 0)),
        ],
        out_specs=pl.BlockSpec((TT, D), lambda t: (t, 0)),
        out_shape=jax.ShapeDtypeStruct((T, D), x.dtype),
        compiler_params=pltpu.CompilerParams(
            dimension_semantics=("parallel",)),
    )(x, gate_w, gate_b.reshape(E, 1), W1, W2, W3, sw1, sw2, sw3)
